# Initial kernel scaffold; baseline (speedup 1.0000x reference)
import jax
import jax.numpy as jnp
from jax.experimental import pallas as pl


def kernel(user_id, item_id, category, table_user_id, table_item_id, table_category):
    raise NotImplementedError("write your pallas kernel here")


# speedup vs baseline: 2.3192x; 2.3192x over previous
import functools

import jax
import jax.numpy as jnp
from jax import lax
from jax.experimental import pallas as pl
from jax.experimental.pallas import tpu as pltpu
from jax.experimental.pallas import tpu_sc as plsc

EMBED_DIM = 128
BATCH = 16384

_info = plsc.get_sparse_core_info()
NUM_CORES = _info.num_cores
NUM_SUBCORES = _info.num_subcores
NUM_WORKERS = NUM_CORES * NUM_SUBCORES
B_PER_W = BATCH // NUM_WORKERS
CHUNK = 128
NCHUNK = B_PER_W // CHUNK
NFEAT = 3


@functools.partial(
    pl.kernel,
    mesh=plsc.VectorSubcoreMesh(core_axis_name="c", subcore_axis_name="s"),
    out_type=[jax.ShapeDtypeStruct((BATCH, EMBED_DIM), jnp.float32)] * NFEAT,
    scratch_types=[
        pltpu.VMEM((NFEAT, NCHUNK, CHUNK), jnp.int32),
        pltpu.VMEM((2, CHUNK, EMBED_DIM), jnp.float32),
        pltpu.SemaphoreType.DMA,
        pltpu.SemaphoreType.DMA,
        pltpu.SemaphoreType.DMA,
        pltpu.SemaphoreType.DMA,
    ],
)
def _lookup3(idx_u, idx_i, idx_c, tab_u, tab_i, tab_c,
             out_u, out_i, out_c,
             idx_v, rows_v, gsem0, gsem1, ssem0, ssem1):
    wid = lax.axis_index("s") * NUM_CORES + lax.axis_index("c")
    base = wid * B_PER_W

    idx_hbm = [idx_u, idx_i, idx_c]
    tabs = [tab_u, tab_i, tab_c]
    outs = [out_u, out_i, out_c]
    gsems = [gsem0, gsem1]
    ssems = [ssem0, ssem1]

    for f in range(NFEAT):
        pltpu.sync_copy(idx_hbm[f].at[wid], idx_v.at[f])

    chunks = [(f, j) for f in range(NFEAT) for j in range(NCHUNK)]
    n = len(chunks)

    def gather_start(k, b):
        f, j = chunks[k]
        return pltpu.async_copy(tabs[f].at[idx_v.at[f, j]], rows_v.at[b],
                                gsems[b])

    def store_start(k, b):
        f, j = chunks[k]
        return pltpu.async_copy(rows_v.at[b],
                                outs[f].at[pl.ds(base + j * CHUNK, CHUNK)],
                                ssems[b])

    g = [None, None]
    s = [None, None]
    g[0] = gather_start(0, 0)
    for k in range(n):
        b = k % 2
        g[b].wait()
        if k + 1 < n:
            nb = (k + 1) % 2
            if s[nb] is not None:
                s[nb].wait()
            g[nb] = gather_start(k + 1, nb)
        s[b] = store_start(k, b)
    s[0].wait()
    s[1].wait()


def kernel(user_id, item_id, category, table_user_id, table_item_id,
           table_category):
    idx = [
        x.reshape(NUM_WORKERS, NCHUNK, CHUNK)
        for x in (user_id, item_id, category)
    ]
    out = _lookup3(idx[0], idx[1], idx[2],
                   table_user_id, table_item_id, table_category)
    return tuple(out)


# speedup vs baseline: 2.5247x; 1.0886x over previous
import functools

import jax
import jax.numpy as jnp
from jax import lax
from jax.experimental import pallas as pl
from jax.experimental.pallas import tpu as pltpu
from jax.experimental.pallas import tpu_sc as plsc

EMBED_DIM = 128
BATCH = 16384

_info = plsc.get_sparse_core_info()
NUM_CORES = _info.num_cores
NUM_SUBCORES = _info.num_subcores
NUM_WORKERS = NUM_CORES * NUM_SUBCORES
B_PER_W = BATCH // NUM_WORKERS
CHUNK = 128
NCHUNK = B_PER_W // CHUNK
NFEAT = 3


@functools.partial(
    pl.kernel,
    mesh=plsc.VectorSubcoreMesh(core_axis_name="c", subcore_axis_name="s"),
    out_type=[jax.ShapeDtypeStruct((BATCH, EMBED_DIM), jnp.float32)] * NFEAT,
    scratch_types=[
        pltpu.VMEM((NFEAT, NCHUNK, CHUNK), jnp.int32),
        pltpu.VMEM((4, CHUNK, EMBED_DIM), jnp.float32),
    ] + [pltpu.SemaphoreType.DMA] * 8,
)
def _lookup3(idx_u, idx_i, idx_c, tab_u, tab_i, tab_c,
             out_u, out_i, out_c,
             idx_v, rows_v, *sems):
    wid = lax.axis_index("s") * NUM_CORES + lax.axis_index("c")
    base = wid * B_PER_W

    idx_hbm = [idx_u, idx_i, idx_c]
    tabs = [tab_u, tab_i, tab_c]
    outs = [out_u, out_i, out_c]
    NBUF = 4
    DRAIN_LAG = 2
    gsems = sems[:NBUF]
    ssems = sems[NBUF:]

    for f in range(NFEAT):
        pltpu.sync_copy(idx_hbm[f].at[wid], idx_v.at[f])

    chunks = [(f, j) for f in range(NFEAT) for j in range(NCHUNK)]
    n = len(chunks)

    def gather_start(k, b):
        f, j = chunks[k]
        return pltpu.async_copy(tabs[f].at[idx_v.at[f, j]], rows_v.at[b],
                                gsems[b])

    def store_start(k, b):
        f, j = chunks[k]
        return pltpu.async_copy(rows_v.at[b],
                                outs[f].at[pl.ds(base + j * CHUNK, CHUNK)],
                                ssems[b])

    g = [None] * NBUF
    s = [None] * NBUF
    for k in range(n + DRAIN_LAG):
        if k < n:
            b = k % NBUF
            if s[b] is not None:
                s[b].wait()
            g[b] = gather_start(k, b)
        d = k - DRAIN_LAG
        if d >= 0:
            bb = d % NBUF
            g[bb].wait()
            s[bb] = store_start(d, bb)
    for b in range(NBUF):
        if s[b] is not None:
            s[b].wait()


def kernel(user_id, item_id, category, table_user_id, table_item_id,
           table_category):
    idx = [
        x.reshape(NUM_WORKERS, NCHUNK, CHUNK)
        for x in (user_id, item_id, category)
    ]
    out = _lookup3(idx[0], idx[1], idx[2],
                   table_user_id, table_item_id, table_category)
    return tuple(out)
